# baseline (speedup 1.0000x reference)
import jax
import jax.numpy as jnp
from jax import lax
from jax.experimental import pallas as pl
from jax.experimental.pallas import tpu as pltpu
from jax.experimental.pallas import tpu_sc as plsc

NUM_SCIENTISTS = 100000
NUM_PAPERS = 1000000
D = 64
GLOBAL_MEAN = 3.5
IMPLICIT_WEIGHT = 0.5
B = 4096
L = 50

NC, NS = 2, 16
NW = NC * NS
E = B // NW
C = 16
NCH = E // C
DV = D // 16

_LANE_IOTA = None


def _vgather(x, idx):
    dn = lax.GatherDimensionNumbers(
        offset_dims=(), collapsed_slice_dims=(0,), start_index_map=(0,))
    return lax.gather(x, idx[:, None], dn, (1,),
                      mode=lax.GatherScatterMode.PROMISE_IN_BOUNDS)


def _splat(x, i):
    return _vgather(x, jnp.full((16,), i, dtype=jnp.int32))


def _lanesum(t, lane):
    for k in (8, 4, 2, 1):
        t = t + _vgather(t, lane ^ jnp.int32(k))
    return t


def _sc_body(sids_hbm, pids_hbm, imp_hbm, lens_hbm, p_hbm, q_hbm,
             bs_hbm, bp_hbm, out_hbm,
             sids_v, pids_v, lens_v, imp_idx_v, imp_rows_v,
             p_rows_v, q_rows_v, bs_v, bp_v, out_v, sem):
    cid = lax.axis_index("c")
    scid = lax.axis_index("s")
    wid = scid * NC + cid

    pltpu.sync_copy(sids_hbm.at[wid], sids_v)
    pltpu.sync_copy(pids_hbm.at[wid], pids_v)
    pltpu.sync_copy(lens_hbm.at[wid], lens_v)
    pltpu.sync_copy(imp_hbm.at[wid], imp_idx_v)

    lane = lax.iota(jnp.int32, 16)

    def chunk(ch, carry):
        cps = []
        for i in range(C):
            cps.append(pltpu.async_copy(
                q_hbm.at[imp_idx_v.at[ch, i]], imp_rows_v.at[i], sem))
        cps.append(pltpu.async_copy(p_hbm.at[sids_v.at[ch]], p_rows_v, sem))
        cps.append(pltpu.async_copy(q_hbm.at[pids_v.at[ch]], q_rows_v, sem))
        cps.append(pltpu.async_copy(bs_hbm.at[sids_v.at[ch]], bs_v, sem))
        cps.append(pltpu.async_copy(bp_hbm.at[pids_v.at[ch]], bp_v, sem))
        for cp in cps:
            cp.wait()

        lens = lens_v[ch, :]
        lens_f = lens.astype(jnp.float32)
        h = 0.5 * lens_f
        yb = jnp.int32(0x5F3759DF) - (lax.bitcast_convert_type(
            lens_f, jnp.int32) >> 1)
        y = lax.bitcast_convert_type(yb, jnp.float32)
        for _ in range(3):
            y = y * (1.5 - h * y * y)
        sqrt_n = lens_f * y
        alpha = IMPLICIT_WEIGHT / (sqrt_n + 1e-9)

        def elem(i, out_vec):
            len_i = _splat(lens, i)
            a_i = _splat(alpha, i)
            zero = jnp.zeros((16,), jnp.float32)
            one = jnp.int32(1)
            zeroi = jnp.int32(0)
            acc = [zero, zero, zero, zero]
            for l in range(L):
                mf = jnp.minimum(jnp.maximum(len_i - jnp.int32(l), zeroi),
                                 one).astype(jnp.float32)
                for d in range(DV):
                    acc[d] = acc[d] + mf * imp_rows_v[i, l, pl.ds(d * 16, 16)]
            t = zero
            for d in range(DV):
                u = p_rows_v[i, pl.ds(d * 16, 16)] + a_i * acc[d]
                t = t + q_rows_v[i, pl.ds(d * 16, 16)] * u
            tot = _lanesum(t, lane)
            eq = jnp.minimum(jnp.maximum(one - jnp.abs(lane - i), zeroi),
                             one).astype(jnp.float32)
            return out_vec + eq * tot

        out_vec = lax.fori_loop(0, C, elem, jnp.zeros((16,), jnp.float32))
        out_v[ch, :] = GLOBAL_MEAN + bs_v[:] + bp_v[:] + out_vec
        return carry

    lax.fori_loop(0, NCH, chunk, jnp.int32(0))
    pltpu.sync_copy(out_v, out_hbm.at[wid])


def kernel(SIDs, PIDs, implicit_PIDs, implicit_lengths, P, Q,
           scientist_bias, paper_bias):
    mesh = plsc.VectorSubcoreMesh(core_axis_name="c", subcore_axis_name="s",
                                  num_cores=NC, num_subcores=NS)
    run = pl.kernel(
        _sc_body,
        out_type=jax.ShapeDtypeStruct((NW, NCH, C), jnp.float32),
        mesh=mesh,
        compiler_params=pltpu.CompilerParams(use_tc_tiling_on_sc=False),
        scratch_types=[
            pltpu.VMEM((NCH, C), jnp.int32),
            pltpu.VMEM((NCH, C), jnp.int32),
            pltpu.VMEM((NCH, C), jnp.int32),
            pltpu.VMEM((NCH, C, L), jnp.int32),
            pltpu.VMEM((C, L, D), jnp.float32),
            pltpu.VMEM((C, D), jnp.float32),
            pltpu.VMEM((C, D), jnp.float32),
            pltpu.VMEM((C,), jnp.float32),
            pltpu.VMEM((C,), jnp.float32),
            pltpu.VMEM((NCH, C), jnp.float32),
            pltpu.SemaphoreType.DMA,
        ],
    )
    out = run(
        SIDs.reshape(NW, NCH, C).astype(jnp.int32),
        PIDs.reshape(NW, NCH, C).astype(jnp.int32),
        implicit_PIDs.reshape(NW, NCH, C, L).astype(jnp.int32),
        implicit_lengths.reshape(NW, NCH, C).astype(jnp.int32),
        P,
        Q,
        scientist_bias.reshape(NUM_SCIENTISTS),
        paper_bias.reshape(NUM_PAPERS),
    )
    return out.reshape(B)
